# baseline (speedup 1.0000x reference)
import functools

import jax
import jax.numpy as jnp
from jax import lax
from jax.experimental import pallas as pl
from jax.experimental.pallas import tpu as pltpu
from jax.experimental.pallas import tpu_sc as plsc

B = 4096
S = 200
D = 64
V = 1000000
NC = 2
NS = 16
NW = NC * NS
RPW = B // NW
C0, C1 = 128, S - 128
NBLK = V // 128
VTAIL = NBLK * 128
DC = D // 2
UNROLL = 8
INV_S = 1.0 / S

_mesh = plsc.VectorSubcoreMesh(core_axis_name="c", subcore_axis_name="s")


@functools.partial(
    pl.kernel,
    out_type=jax.ShapeDtypeStruct((V // 4, 128), jnp.float32),
    mesh=_mesh,
    scratch_types=[
        pltpu.VMEM((2, D, 129), jnp.float32),
        pltpu.VMEM((2, 32, 128), jnp.float32),
        pltpu.VMEM((D, 64), jnp.float32),
        pltpu.VMEM((16, 128), jnp.float32),
        pltpu.VMEM((DC, 129), jnp.float32),
        pltpu.SemaphoreType.DMA,
        pltpu.SemaphoreType.DMA,
        pltpu.SemaphoreType.DMA,
        pltpu.SemaphoreType.DMA,
    ],
    compiler_params=pltpu.CompilerParams(needs_layout_passes=False),
)
def _repack(tT_hbm, w_hbm, in_v, out_v, tin_v, tout_v, pk_v, si0, si1, so0, so1):
    wid = lax.axis_index("s") * NC + lax.axis_index("c")
    sis = (si0, si1)
    sos = (so0, so1)

    def in_copy(tb, buf):
        off = pl.multiple_of(tb, 128)
        return pltpu.make_async_copy(
            tT_hbm.at[:, pl.ds(off, 128)],
            in_v.at[buf].at[:, pl.ds(0, 128)], sis[buf])

    def out_copy(tb, buf):
        off = pl.multiple_of(tb // 4, 32)
        return pltpu.make_async_copy(
            out_v.at[buf], w_hbm.at[pl.ds(off, 32)], sos[buf])

    row16 = lax.iota(jnp.int32, 16)
    rnd = jnp.full((16,), 0x8000, jnp.int32)
    himask = jnp.full((16,), -65536, jnp.int32)

    def bf16_pair(lo_f32, hi_f32):
        li = plsc.bitcast(lo_f32, jnp.int32)
        hi = plsc.bitcast(hi_f32, jnp.int32)
        lo = lax.shift_right_logical(li + rnd, jnp.full((16,), 16, jnp.int32))
        return plsc.bitcast(((hi + rnd) & himask) | lo, jnp.float32)

    def transpose_pack(src, dst, n_tracks):
        @plsc.parallel_loop(0, DC, step=1, unroll=8)
        def _(j):
            lo = j + (j & 16)
            for g in range(n_tracks // 16):
                pk = bf16_pair(
                    src[lo, pl.ds(16 * g, 16)], src[lo + 16, pl.ds(16 * g, 16)])
                pos = (row16 + 16 * g) * 32 + j
                plsc.store_scatter(
                    dst,
                    [lax.shift_right_logical(pos, jnp.full((16,), 7, jnp.int32)),
                     pos & jnp.full((16,), 127, jnp.int32)],
                    pk)

    @pl.when(wid == NW - 1)
    def _():
        pltpu.sync_copy(tT_hbm.at[:, pl.ds(VTAIL, V - VTAIL)], tin_v)
        transpose_pack(tin_v, tout_v, V - VTAIL)
        pltpu.sync_copy(tout_v, w_hbm.at[pl.ds(VTAIL // 4, (V - VTAIL) // 4)])

    NM = (NBLK + NW - 1) // NW

    def blk(i):
        return jnp.minimum(wid + i * NW, NBLK - 1) * 128

    in_copy(blk(0), 0).start()

    @pl.loop(0, NM, step=2)
    def _(g):
        for off in (0, 1):
            i = g + off
            buf = off

            @pl.when(i < NM)
            def _():
                @pl.when(i + 1 < NM)
                def _():
                    in_copy(blk(i + 1), 1 - buf).start()

                in_copy(blk(i), buf).wait()

                @pl.when(i >= 2)
                def _():
                    out_copy(blk(i - 2), buf).wait()

                transpose_pack(in_v.at[buf], out_v.at[buf], 128)
                out_copy(blk(i), buf).start()

    out_copy(blk(NM - 2), (NM - 2) % 2).wait()
    out_copy(blk(NM - 1), (NM - 1) % 2).wait()


@functools.partial(
    pl.kernel,
    out_type=jax.ShapeDtypeStruct((B, D), jnp.float32),
    mesh=_mesh,
    scratch_types=[
        pltpu.VMEM((RPW, S), jnp.int32),
        pltpu.VMEM((2, S, DC), jnp.float32),
        pltpu.VMEM((RPW, D), jnp.float32),
        pltpu.SemaphoreType.DMA,
        pltpu.SemaphoreType.DMA,
    ],
    compiler_params=pltpu.CompilerParams(
        use_tc_tiling_on_sc=False, needs_layout_passes=False),
)
def _gather_mean(z_hbm, x_hbm, out_hbm, idx_v, rows_v, out_v, sem0, sem1):
    wid = lax.axis_index("s") * NC + lax.axis_index("c")
    base = wid * RPW

    pltpu.sync_copy(x_hbm.at[pl.ds(base, RPW)], idx_v)

    def _copies(r, buf, sem):
        row_idx = idx_v.at[r]
        c0 = pltpu.make_async_copy(
            z_hbm.at[row_idx.at[pl.ds(0, C0)]],
            rows_v.at[buf].at[pl.ds(0, C0)], sem)
        c1 = pltpu.make_async_copy(
            z_hbm.at[row_idx.at[pl.ds(C0, C1)]],
            rows_v.at[buf].at[pl.ds(C0, C1)], sem)
        return c0, c1

    def fire(r, buf, sem):
        for c in _copies(r, buf, sem):
            c.start()

    def drain(r, buf, sem):
        for c in _copies(r, buf, sem):
            c.wait()

    shift16 = jnp.full((16,), 16, jnp.int32)
    himask = jnp.full((16,), -65536, jnp.int32)

    def unpack_pair(container_f32):
        ci = plsc.bitcast(container_f32, jnp.int32)
        lo = plsc.bitcast(lax.shift_left(ci, shift16), jnp.float32)
        hi = plsc.bitcast(ci & himask, jnp.float32)
        return lo, hi

    def accum_store(r, buf):
        rows = rows_v.at[buf]
        zero = jnp.zeros((16,), jnp.float32)

        def body(i, acc):
            a = list(acc)
            s0 = i * UNROLL
            for u in range(UNROLL):
                v0, v1 = unpack_pair(rows[s0 + u, pl.ds(0, 16)])
                v2, v3 = unpack_pair(rows[s0 + u, pl.ds(16, 16)])
                k = (u % 2) * 4
                a[k] = a[k] + v0
                a[k + 1] = a[k + 1] + v1
                a[k + 2] = a[k + 2] + v2
                a[k + 3] = a[k + 3] + v3
            return tuple(a)

        acc = lax.fori_loop(0, S // UNROLL, body, (zero,) * 8)
        for c in range(4):
            out_v[r, pl.ds(c * 16, 16)] = (acc[c] + acc[4 + c]) * INV_S

    fire(0, 0, sem0)

    @pl.loop(0, RPW, step=2)
    def _(g):
        fire(g + 1, 1, sem1)
        drain(g, 0, sem0)
        accum_store(g, 0)

        @pl.when(g + 2 < RPW)
        def _():
            fire(g + 2, 0, sem0)

        drain(g + 1, 1, sem1)
        accum_store(g + 1, 1)

    pltpu.sync_copy(out_v, out_hbm.at[pl.ds(base, RPW)])


def kernel(x, table):
    w = _repack(table.T)
    return _gather_mean(w.reshape(V, DC), x)


# speedup vs baseline: 1.0022x; 1.0022x over previous
import functools

import jax
import jax.numpy as jnp
from jax import lax
from jax.experimental import pallas as pl
from jax.experimental.pallas import tpu as pltpu
from jax.experimental.pallas import tpu_sc as plsc

B = 4096
S = 200
D = 64
V = 1000000
NC = 2
NS = 16
NW = NC * NS
RPW = B // NW
C0, C1 = 128, S - 128
NBLK = V // 128
VTAIL = NBLK * 128
DC = D // 2
UNROLL = 8
INV_S = 1.0 / S

_mesh = plsc.VectorSubcoreMesh(core_axis_name="c", subcore_axis_name="s")


@functools.partial(
    pl.kernel,
    out_type=jax.ShapeDtypeStruct((V // 4, 128), jnp.float32),
    mesh=_mesh,
    scratch_types=[
        pltpu.VMEM((2, D, 129), jnp.float32),
        pltpu.VMEM((2, 32, 128), jnp.float32),
        pltpu.VMEM((D, 64), jnp.float32),
        pltpu.VMEM((16, 128), jnp.float32),
        pltpu.SemaphoreType.DMA,
        pltpu.SemaphoreType.DMA,
        pltpu.SemaphoreType.DMA,
        pltpu.SemaphoreType.DMA,
    ],
    compiler_params=pltpu.CompilerParams(needs_layout_passes=False),
)
def _repack(tT_hbm, w_hbm, in_v, out_v, tin_v, tout_v, si0, si1, so0, so1):
    wid = lax.axis_index("s") * NC + lax.axis_index("c")
    sis = (si0, si1)
    sos = (so0, so1)

    def in_copy(tb, buf):
        off = pl.multiple_of(tb, 128)
        return pltpu.make_async_copy(
            tT_hbm.at[:, pl.ds(off, 128)],
            in_v.at[buf].at[:, pl.ds(0, 128)], sis[buf])

    def out_copy(tb, buf):
        off = pl.multiple_of(tb // 4, 32)
        return pltpu.make_async_copy(
            out_v.at[buf], w_hbm.at[pl.ds(off, 32)], sos[buf])

    row16 = lax.iota(jnp.int32, 16)
    rnd = jnp.full((16,), 0x8000, jnp.int32)
    himask = jnp.full((16,), -65536, jnp.int32)

    def bf16_pair(lo_f32, hi_f32):
        li = plsc.bitcast(lo_f32, jnp.int32)
        hi = plsc.bitcast(hi_f32, jnp.int32)
        lo = lax.shift_right_logical(li + rnd, jnp.full((16,), 16, jnp.int32))
        return plsc.bitcast(((hi + rnd) & himask) | lo, jnp.float32)

    def transpose_pack(src, dst, n_tracks):
        @plsc.parallel_loop(0, DC, step=1, unroll=8)
        def _(j):
            lo = j + (j & 16)
            for g in range(n_tracks // 16):
                pk = bf16_pair(
                    src[lo, pl.ds(16 * g, 16)], src[lo + 16, pl.ds(16 * g, 16)])
                pos = (row16 + 16 * g) * 32 + j
                plsc.store_scatter(
                    dst,
                    [lax.shift_right_logical(pos, jnp.full((16,), 7, jnp.int32)),
                     pos & jnp.full((16,), 127, jnp.int32)],
                    pk)

    @pl.when(wid == NW - 1)
    def _():
        pltpu.sync_copy(tT_hbm.at[:, pl.ds(VTAIL, V - VTAIL)], tin_v)
        transpose_pack(tin_v, tout_v, V - VTAIL)
        pltpu.sync_copy(tout_v, w_hbm.at[pl.ds(VTAIL // 4, (V - VTAIL) // 4)])

    NM = (NBLK + NW - 1) // NW

    def blk(i):
        return jnp.minimum(wid + i * NW, NBLK - 1) * 128

    in_copy(blk(0), 0).start()

    @pl.loop(0, NM, step=2)
    def _(g):
        for off in (0, 1):
            i = g + off
            buf = off

            @pl.when(i < NM)
            def _():
                @pl.when(i + 1 < NM)
                def _():
                    in_copy(blk(i + 1), 1 - buf).start()

                in_copy(blk(i), buf).wait()

                @pl.when(i >= 2)
                def _():
                    out_copy(blk(i - 2), buf).wait()

                transpose_pack(in_v.at[buf], out_v.at[buf], 128)
                out_copy(blk(i), buf).start()

    out_copy(blk(NM - 2), (NM - 2) % 2).wait()
    out_copy(blk(NM - 1), (NM - 1) % 2).wait()


@functools.partial(
    pl.kernel,
    out_type=jax.ShapeDtypeStruct((B, D), jnp.float32),
    mesh=_mesh,
    scratch_types=[
        pltpu.VMEM((RPW, S), jnp.int32),
        pltpu.VMEM((2, S, DC), jnp.float32),
        pltpu.VMEM((RPW, D), jnp.float32),
        pltpu.SemaphoreType.DMA,
        pltpu.SemaphoreType.DMA,
    ],
    compiler_params=pltpu.CompilerParams(
        use_tc_tiling_on_sc=False, needs_layout_passes=False),
)
def _gather_mean(z_hbm, x_hbm, out_hbm, idx_v, rows_v, out_v, sem0, sem1):
    wid = lax.axis_index("s") * NC + lax.axis_index("c")
    base = wid * RPW

    pltpu.sync_copy(x_hbm.at[pl.ds(base, RPW)], idx_v)

    def _copies(r, buf, sem):
        row_idx = idx_v.at[r]
        c0 = pltpu.make_async_copy(
            z_hbm.at[row_idx.at[pl.ds(0, C0)]],
            rows_v.at[buf].at[pl.ds(0, C0)], sem)
        c1 = pltpu.make_async_copy(
            z_hbm.at[row_idx.at[pl.ds(C0, C1)]],
            rows_v.at[buf].at[pl.ds(C0, C1)], sem)
        return c0, c1

    def fire(r, buf, sem):
        for c in _copies(r, buf, sem):
            c.start()

    def drain(r, buf, sem):
        for c in _copies(r, buf, sem):
            c.wait()

    shift16 = jnp.full((16,), 16, jnp.int32)
    himask = jnp.full((16,), -65536, jnp.int32)

    def unpack_pair(container_f32):
        ci = plsc.bitcast(container_f32, jnp.int32)
        lo = plsc.bitcast(lax.shift_left(ci, shift16), jnp.float32)
        hi = plsc.bitcast(ci & himask, jnp.float32)
        return lo, hi

    def accum_store(r, buf):
        rows = rows_v.at[buf]
        zero = jnp.zeros((16,), jnp.float32)

        def body(i, acc):
            a = list(acc)
            s0 = i * UNROLL
            for u in range(UNROLL):
                v0, v1 = unpack_pair(rows[s0 + u, pl.ds(0, 16)])
                v2, v3 = unpack_pair(rows[s0 + u, pl.ds(16, 16)])
                k = (u % 2) * 4
                a[k] = a[k] + v0
                a[k + 1] = a[k + 1] + v1
                a[k + 2] = a[k + 2] + v2
                a[k + 3] = a[k + 3] + v3
            return tuple(a)

        acc = lax.fori_loop(0, S // UNROLL, body, (zero,) * 8)
        for c in range(4):
            out_v[r, pl.ds(c * 16, 16)] = (acc[c] + acc[4 + c]) * INV_S

    fire(0, 0, sem0)

    @pl.loop(0, RPW, step=2)
    def _(g):
        fire(g + 1, 1, sem1)
        drain(g, 0, sem0)
        accum_store(g, 0)

        @pl.when(g + 2 < RPW)
        def _():
            fire(g + 2, 0, sem0)

        drain(g + 1, 1, sem1)
        accum_store(g + 1, 1)

    pltpu.sync_copy(out_v, out_hbm.at[pl.ds(base, RPW)])


def kernel(x, table):
    w = _repack(table.T)
    return _gather_mean(w.reshape(V, DC), x)
